# Initial kernel scaffold; baseline (speedup 1.0000x reference)
import jax
import jax.numpy as jnp
from jax.experimental import pallas as pl


def kernel(x, edge_index, batch, W1r, W1s, b1, W2r, W2s, b2, W3r, W3s, b3, W4r, W4s, b4, W5r, W5s, b5, gamma, beta, Wl, bl):
    raise NotImplementedError("write your pallas kernel here")


# speedup vs baseline: 2.1111x; 2.1111x over previous
import functools

import jax
import jax.numpy as jnp
from jax import lax
from jax.experimental import pallas as pl
from jax.experimental.pallas import tpu as pltpu
from jax.experimental.pallas import tpu_sc as plsc

N = 10000
D = 128
E = 320000
G = 64
C = 10

NC = 2
NS = 16
NW = NC * NS
K = 128
RB = 320
NPAD = NW * RB
ACC = RB + 4
CPT = 88
EPT = K * CPT

BR = 1000
NBLK = N // BR


def _sc_agg_body(x_hbm, srcp_hbm, dstl_hbm, out_hbm,
                 sidx, didx, rows_v, acc, sem0, sem1):
    c = lax.axis_index("c")
    s = lax.axis_index("s")
    w = s * NC + c
    base = w * EPT
    sems = (sem0, sem1)

    zero = jnp.zeros((16,), jnp.float32)

    def zrow(r, carry):
        for blk in range(8):
            acc[r, pl.ds(blk * 16, 16)] = zero
        return carry

    lax.fori_loop(0, RB, zrow, 0)

    def stage(j, b):
        off = base + j * K
        pltpu.sync_copy(srcp_hbm.at[pl.ds(off, K)],
                        sidx.at[pl.ds(b * K, K)])
        pltpu.sync_copy(dstl_hbm.at[pl.ds(off, K)],
                        didx.at[pl.ds(b * K, K)])
        pltpu.async_copy(x_hbm.at[sidx.at[pl.ds(b * K, K)]],
                         rows_v.at[b], sems[b])

    stage(0, 0)

    def pair(jp, carry):
        j0 = jp * 2
        for b in (0, 1):
            j = j0 + b
            nb = 1 - b

            @pl.when(j + 1 < CPT)
            def _():
                stage(j + 1, nb)

            pltpu.make_async_copy(x_hbm.at[sidx.at[pl.ds(b * K, K)]],
                                  rows_v.at[b], sems[b]).wait()

            def fold16(g, fcarry):
                dv = didx[pl.ds(b * K + g * 16, 16)]
                for lane in range(16):
                    e = g * 16 + lane
                    r = dv[lane]
                    for blk in range(8):
                        sl = pl.ds(blk * 16, 16)
                        plsc.addupdate(acc.at[r, sl], rows_v[b, e, sl])
                return fcarry

            lax.fori_loop(0, K // 16, fold16, 0)
        return carry

    lax.fori_loop(0, CPT // 2, pair, 0)
    pltpu.sync_copy(acc.at[pl.ds(0, RB)], out_hbm.at[pl.ds(w * RB, RB)])


_sc_agg_cache = []


def _get_sc_agg():
    if not _sc_agg_cache:
        mesh = plsc.VectorSubcoreMesh(core_axis_name="c", subcore_axis_name="s",
                                      num_cores=NC, num_subcores=NS)
        _sc_agg_cache.append(pl.kernel(
            _sc_agg_body,
            out_type=jax.ShapeDtypeStruct((NPAD, D), jnp.float32),
            mesh=mesh,
            scratch_types=[
                pltpu.VMEM((2 * K,), jnp.int32),
                pltpu.VMEM((2 * K,), jnp.int32),
                pltpu.VMEM((2, K, D), jnp.float32),
                pltpu.VMEM((ACC, D), jnp.float32),
                pltpu.SemaphoreType.DMA,
                pltpu.SemaphoreType.DMA,
            ],
        ))
    return _sc_agg_cache[0]


def _layer_first_body(p_ref, x_ref, Wr_ref, Ws_ref, b_ref, xo_ref):
    a = p_ref[...]
    h = jnp.dot(a, Wr_ref[...], preferred_element_type=jnp.float32)
    h += jnp.dot(x_ref[...], Ws_ref[...], preferred_element_type=jnp.float32)
    xo_ref[...] = jnp.maximum(h + b_ref[...], 0.0)


def _layer_mid_body(init_z, p_ref, x_ref, z_ref, Wr_ref, Ws_ref, b_ref,
                    Ar_ref, As_ref, b5_ref, xo_ref, zo_ref):
    a = p_ref[...]
    x = x_ref[...]
    h = jnp.dot(a, Wr_ref[...], preferred_element_type=jnp.float32)
    h += jnp.dot(x, Ws_ref[...], preferred_element_type=jnp.float32)
    xo_ref[...] = jnp.maximum(h + b_ref[...], 0.0)
    zc = jnp.dot(a, Ar_ref[...], preferred_element_type=jnp.float32)
    zc += jnp.dot(x, As_ref[...], preferred_element_type=jnp.float32)
    if init_z:
        zo_ref[...] = zc + b5_ref[...]
    else:
        zo_ref[...] = zc + z_ref[...]


_wspec = pl.BlockSpec((D, D), lambda i: (0, 0))
_bspec = pl.BlockSpec((1, D), lambda i: (0, 0))
_xspec = pl.BlockSpec((BR, D), lambda i: (i, 0))

_layer_first = pl.pallas_call(
    _layer_first_body,
    grid=(NBLK,),
    in_specs=[_xspec, _xspec, _wspec, _wspec, _bspec],
    out_specs=_xspec,
    out_shape=jax.ShapeDtypeStruct((N, D), jnp.float32),
)

_layer_init = pl.pallas_call(
    functools.partial(_layer_mid_body, True),
    grid=(NBLK,),
    in_specs=[_xspec, _xspec, _xspec, _wspec, _wspec, _bspec,
              _wspec, _wspec, _bspec],
    out_specs=[_xspec, _xspec],
    out_shape=[jax.ShapeDtypeStruct((N, D), jnp.float32),
               jax.ShapeDtypeStruct((N, D), jnp.float32)],
)

_layer_mid = pl.pallas_call(
    functools.partial(_layer_mid_body, False),
    grid=(NBLK,),
    in_specs=[_xspec, _xspec, _xspec, _wspec, _wspec, _bspec,
              _wspec, _wspec, _bspec],
    out_specs=[_xspec, _xspec],
    out_shape=[jax.ShapeDtypeStruct((N, D), jnp.float32),
               jax.ShapeDtypeStruct((N, D), jnp.float32)],
)


def _tail_body(p_ref, x_ref, z_ref, Ar_ref, As_ref, gamma_ref, beta_ref,
               Wl_ref, bl_ref, batch_ref, out_ref):
    a = p_ref[...][:N]
    x = x_ref[...]
    x5 = z_ref[...]
    x5 += jnp.dot(a, Ar_ref[...], preferred_element_type=jnp.float32)
    x5 += jnp.dot(x, As_ref[...], preferred_element_type=jnp.float32)
    mu = jnp.mean(x5, axis=0, keepdims=True)
    d = x5 - mu
    var = jnp.mean(d * d, axis=0, keepdims=True)
    x5 = (x5 - mu) * jax.lax.rsqrt(var + 1e-5) * gamma_ref[...] + beta_ref[...]
    seg = jax.lax.broadcasted_iota(jnp.int32, (N, G), 1)
    onehot = (batch_ref[...] == seg).astype(jnp.float32)
    sums = jax.lax.dot_general(onehot, x5, (((0,), (0,)), ((), ())),
                               preferred_element_type=jnp.float32)
    cnt = jnp.sum(onehot, axis=0)[:, None]
    pooled = sums / jnp.maximum(cnt, 1.0)
    out_ref[...] = jnp.dot(pooled, Wl_ref[...],
                           preferred_element_type=jnp.float32) + bl_ref[...]


_tail = pl.pallas_call(
    _tail_body,
    out_shape=jax.ShapeDtypeStruct((G, C), jnp.float32),
)


def kernel(x, edge_index, batch, W1r, W1s, b1, W2r, W2s, b2, W3r, W3s, b3,
           W4r, W4s, b4, W5r, W5s, b5, gamma, beta, Wl, bl):
    src = edge_index[0]
    dst = edge_index[1]
    order = jnp.argsort(dst, stable=True)
    src_s = jnp.take(src, order)
    dst_s = jnp.take(dst, order)
    w_e = dst_s // RB
    starts = jnp.searchsorted(dst_s, (jnp.arange(NW) * RB).astype(jnp.int32))
    pos = jnp.arange(E, dtype=jnp.int32) - jnp.take(starts, w_e).astype(jnp.int32)
    flat = w_e * EPT + pos
    fill = jnp.arange(NW * EPT, dtype=jnp.int32)
    srcp = (fill % N).at[flat].set(src_s)
    dstl = (RB + (fill % 4)).at[flat].set(dst_s - w_e * RB)

    b1r = b1.reshape(1, D)
    b2r = b2.reshape(1, D)
    b3r = b3.reshape(1, D)
    b4r = b4.reshape(1, D)
    b5r = b5.reshape(1, D)
    W5r_ = [W5r[i * D:(i + 1) * D] for i in range(4)]
    W5s_ = [W5s[i * D:(i + 1) * D] for i in range(4)]

    _sc_agg = _get_sc_agg()
    p0 = _sc_agg(x, srcp, dstl)
    x1 = _layer_first(p0, x, W1r, W1s, b1r)
    p1 = _sc_agg(x1, srcp, dstl)
    x2, z = _layer_init(p1, x1, x1, W2r, W2s, b2r, W5r_[0], W5s_[0], b5r)
    p2 = _sc_agg(x2, srcp, dstl)
    x3, z = _layer_mid(p2, x2, z, W3r, W3s, b3r, W5r_[1], W5s_[1], b5r)
    p3 = _sc_agg(x3, srcp, dstl)
    x4, z = _layer_mid(p3, x3, z, W4r, W4s, b4r, W5r_[2], W5s_[2], b5r)
    p4 = _sc_agg(x4, srcp, dstl)
    return _tail(p4, x4, z, W5r_[3], W5s_[3], gamma, beta, Wl, bl,
                 batch.reshape(N, 1))


# speedup vs baseline: 3.8385x; 1.8183x over previous
import functools

import jax
import jax.numpy as jnp
from jax import lax
from jax.experimental import pallas as pl
from jax.experimental.pallas import tpu as pltpu
from jax.experimental.pallas import tpu_sc as plsc

N = 10000
D = 128
E = 320000
G = 64
C = 10

NC = 2
NS = 16
NW = NC * NS
K = 128
RB = 320
NPAD = NW * RB
ACC = RB + 4
CPT = 88
EPT = K * CPT

BR = 1000
NBLK = N // BR


def _sc_agg_body(x_hbm, srcp_hbm, dstl_hbm, out_hbm,
                 sidx, didx, rows_v, acc, sem0, sem1):
    c = lax.axis_index("c")
    s = lax.axis_index("s")
    w = s * NC + c
    base = w * EPT
    sems = (sem0, sem1)

    zero = jnp.zeros((16,), jnp.float32)

    def zrow(r, carry):
        for blk in range(8):
            acc[r, pl.ds(blk * 16, 16)] = zero
        return carry

    lax.fori_loop(0, RB, zrow, 0)

    def stage(j, b):
        off = base + j * K
        pltpu.sync_copy(srcp_hbm.at[pl.ds(off, K)],
                        sidx.at[pl.ds(b * K, K)])
        pltpu.sync_copy(dstl_hbm.at[pl.ds(off, K)],
                        didx.at[pl.ds(b * K, K)])
        pltpu.async_copy(x_hbm.at[sidx.at[pl.ds(b * K, K)]],
                         rows_v.at[b], sems[b])

    stage(0, 0)

    def pair(jp, carry):
        j0 = jp * 2
        for b in (0, 1):
            j = j0 + b
            nb = 1 - b

            @pl.when(j + 1 < CPT)
            def _():
                stage(j + 1, nb)

            pltpu.make_async_copy(x_hbm.at[sidx.at[pl.ds(b * K, K)]],
                                  rows_v.at[b], sems[b]).wait()

            def fold16(g, fcarry):
                dv = didx[pl.ds(b * K + g * 16, 16)]
                for lane in range(16):
                    e = g * 16 + lane
                    r = dv[lane]
                    for blk in range(8):
                        sl = pl.ds(blk * 16, 16)
                        plsc.addupdate(acc.at[r, sl], rows_v[b, e, sl])
                return fcarry

            lax.fori_loop(0, K // 16, fold16, 0)
        return carry

    lax.fori_loop(0, CPT // 2, pair, 0)
    pltpu.sync_copy(acc.at[pl.ds(0, RB)], out_hbm.at[pl.ds(w * RB, RB)])


_sc_agg_cache = []


def _get_sc_agg():
    if not _sc_agg_cache:
        mesh = plsc.VectorSubcoreMesh(core_axis_name="c", subcore_axis_name="s",
                                      num_cores=NC, num_subcores=NS)
        _sc_agg_cache.append(pl.kernel(
            _sc_agg_body,
            out_type=jax.ShapeDtypeStruct((NPAD, D), jnp.float32),
            mesh=mesh,
            scratch_types=[
                pltpu.VMEM((2 * K,), jnp.int32),
                pltpu.VMEM((2 * K,), jnp.int32),
                pltpu.VMEM((2, K, D), jnp.float32),
                pltpu.VMEM((ACC, D), jnp.float32),
                pltpu.SemaphoreType.DMA,
                pltpu.SemaphoreType.DMA,
            ],
        ))
    return _sc_agg_cache[0]


def _layer_first_body(p_ref, x_ref, Wr_ref, Ws_ref, b_ref, xo_ref):
    a = p_ref[...]
    h = jnp.dot(a, Wr_ref[...], preferred_element_type=jnp.float32)
    h += jnp.dot(x_ref[...], Ws_ref[...], preferred_element_type=jnp.float32)
    xo_ref[...] = jnp.maximum(h + b_ref[...], 0.0)


def _layer_mid_body(init_z, p_ref, x_ref, z_ref, Wr_ref, Ws_ref, b_ref,
                    Ar_ref, As_ref, b5_ref, xo_ref, zo_ref):
    a = p_ref[...]
    x = x_ref[...]
    h = jnp.dot(a, Wr_ref[...], preferred_element_type=jnp.float32)
    h += jnp.dot(x, Ws_ref[...], preferred_element_type=jnp.float32)
    xo_ref[...] = jnp.maximum(h + b_ref[...], 0.0)
    zc = jnp.dot(a, Ar_ref[...], preferred_element_type=jnp.float32)
    zc += jnp.dot(x, As_ref[...], preferred_element_type=jnp.float32)
    if init_z:
        zo_ref[...] = zc + b5_ref[...]
    else:
        zo_ref[...] = zc + z_ref[...]


_wspec = pl.BlockSpec((D, D), lambda i: (0, 0))
_bspec = pl.BlockSpec((1, D), lambda i: (0, 0))
_xspec = pl.BlockSpec((BR, D), lambda i: (i, 0))

_layer_first = pl.pallas_call(
    _layer_first_body,
    grid=(NBLK,),
    in_specs=[_xspec, _xspec, _wspec, _wspec, _bspec],
    out_specs=_xspec,
    out_shape=jax.ShapeDtypeStruct((N, D), jnp.float32),
)

_layer_init = pl.pallas_call(
    functools.partial(_layer_mid_body, True),
    grid=(NBLK,),
    in_specs=[_xspec, _xspec, _xspec, _wspec, _wspec, _bspec,
              _wspec, _wspec, _bspec],
    out_specs=[_xspec, _xspec],
    out_shape=[jax.ShapeDtypeStruct((N, D), jnp.float32),
               jax.ShapeDtypeStruct((N, D), jnp.float32)],
)

_layer_mid = pl.pallas_call(
    functools.partial(_layer_mid_body, False),
    grid=(NBLK,),
    in_specs=[_xspec, _xspec, _xspec, _wspec, _wspec, _bspec,
              _wspec, _wspec, _bspec],
    out_specs=[_xspec, _xspec],
    out_shape=[jax.ShapeDtypeStruct((N, D), jnp.float32),
               jax.ShapeDtypeStruct((N, D), jnp.float32)],
)


def _tail_body(p_ref, x_ref, z_ref, Ar_ref, As_ref, gamma_ref, beta_ref,
               Wl_ref, bl_ref, batch_ref, out_ref):
    a = p_ref[...][:N]
    x = x_ref[...]
    x5 = z_ref[...]
    x5 += jnp.dot(a, Ar_ref[...], preferred_element_type=jnp.float32)
    x5 += jnp.dot(x, As_ref[...], preferred_element_type=jnp.float32)
    mu = jnp.mean(x5, axis=0, keepdims=True)
    d = x5 - mu
    var = jnp.mean(d * d, axis=0, keepdims=True)
    x5 = (x5 - mu) * jax.lax.rsqrt(var + 1e-5) * gamma_ref[...] + beta_ref[...]
    seg = jax.lax.broadcasted_iota(jnp.int32, (N, G), 1)
    onehot = (batch_ref[...] == seg).astype(jnp.float32)
    sums = jax.lax.dot_general(onehot, x5, (((0,), (0,)), ((), ())),
                               preferred_element_type=jnp.float32)
    cnt = jnp.sum(onehot, axis=0)[:, None]
    pooled = sums / jnp.maximum(cnt, 1.0)
    out_ref[...] = jnp.dot(pooled, Wl_ref[...],
                           preferred_element_type=jnp.float32) + bl_ref[...]


_tail = pl.pallas_call(
    _tail_body,
    out_shape=jax.ShapeDtypeStruct((G, C), jnp.float32),
)


def kernel(x, edge_index, batch, W1r, W1s, b1, W2r, W2s, b2, W3r, W3s, b3,
           W4r, W4s, b4, W5r, W5s, b5, gamma, beta, Wl, bl):
    src = edge_index[0]
    dst = edge_index[1]
    order = jnp.argsort(dst, stable=True)
    src_s = jnp.take(src, order)
    dst_s = jnp.take(dst, order)
    starts = jnp.searchsorted(dst_s, (jnp.arange(NW + 1) * RB).astype(jnp.int32))
    starts = starts.astype(jnp.int32)
    wgrid = jnp.arange(NW, dtype=jnp.int32)[:, None]
    pgrid = jnp.arange(EPT, dtype=jnp.int32)[None, :]
    e_idx = starts[:NW, None] + pgrid
    valid = e_idx < starts[1:, None]
    e_c = jnp.minimum(e_idx, E - 1)
    flat = wgrid * EPT + pgrid
    srcp = jnp.where(valid, jnp.take(src_s, e_c), flat % N).reshape(-1)
    dstl = jnp.where(valid, jnp.take(dst_s, e_c) - wgrid * RB,
                     RB + (flat & 3)).reshape(-1)

    b1r = b1.reshape(1, D)
    b2r = b2.reshape(1, D)
    b3r = b3.reshape(1, D)
    b4r = b4.reshape(1, D)
    b5r = b5.reshape(1, D)
    W5r_ = [W5r[i * D:(i + 1) * D] for i in range(4)]
    W5s_ = [W5s[i * D:(i + 1) * D] for i in range(4)]

    _sc_agg = _get_sc_agg()
    p0 = _sc_agg(x, srcp, dstl)
    x1 = _layer_first(p0, x, W1r, W1s, b1r)
    p1 = _sc_agg(x1, srcp, dstl)
    x2, z = _layer_init(p1, x1, x1, W2r, W2s, b2r, W5r_[0], W5s_[0], b5r)
    p2 = _sc_agg(x2, srcp, dstl)
    x3, z = _layer_mid(p2, x2, z, W3r, W3s, b3r, W5r_[1], W5s_[1], b5r)
    p3 = _sc_agg(x3, srcp, dstl)
    x4, z = _layer_mid(p3, x3, z, W4r, W4s, b4r, W5r_[2], W5s_[2], b5r)
    p4 = _sc_agg(x4, srcp, dstl)
    return _tail(p4, x4, z, W5r_[3], W5s_[3], gamma, beta, Wl, bl,
                 batch.reshape(N, 1))
